# baseline (speedup 1.0000x reference)
import functools

import jax
import jax.numpy as jnp
from jax import lax
from jax.experimental import pallas as pl
from jax.experimental.pallas import tpu as pltpu
from jax.experimental.pallas import tpu_sc as plsc

NUM_CORES = 2
NUM_SUBCORES = 16
NUM_WORKERS = NUM_CORES * NUM_SUBCORES


@functools.partial(jax.jit, static_argnames=("batch", "dim"))
def _embed_lookup(y, embeddings, batch, dim):
    b_per_w = batch // NUM_WORKERS
    chunk = 32
    n_chunks = b_per_w // chunk

    mesh = plsc.VectorSubcoreMesh(core_axis_name="c", subcore_axis_name="s")

    @functools.partial(
        pl.kernel,
        mesh=mesh,
        out_type=jax.ShapeDtypeStruct((batch, dim), jnp.float32),
        scratch_types=[
            pltpu.VMEM((b_per_w,), jnp.int32),
            pltpu.VMEM((chunk, dim), jnp.float32),
            pltpu.VMEM((chunk, dim), jnp.float32),
            pltpu.SemaphoreType.DMA,
            pltpu.SemaphoreType.DMA,
        ],
    )
    def k(table_hbm, idx_hbm, out_hbm, idx_v, buf0, buf1, gsem, ssem):
        wid = lax.axis_index("s") * NUM_CORES + lax.axis_index("c")
        base = wid * b_per_w
        pltpu.sync_copy(idx_hbm.at[pl.ds(base, b_per_w)], idx_v)

        bufs = (buf0, buf1)

        def gather(c, buf):
            return pltpu.make_async_copy(
                table_hbm.at[idx_v.at[pl.ds(c * chunk, chunk)]], buf, gsem
            )

        def store(c, buf):
            return pltpu.make_async_copy(
                buf, out_hbm.at[pl.ds(base + c * chunk, chunk)], ssem
            )

        gather(0, bufs[0]).start()
        for c in range(n_chunks):
            buf = bufs[c % 2]
            gather(c, buf).wait()
            if c + 1 < n_chunks:
                nxt = bufs[(c + 1) % 2]
                if c >= 1:
                    store(c - 1, nxt).wait()
                gather(c + 1, nxt).start()
            store(c, buf).start()
        store(n_chunks - 2, bufs[(n_chunks - 2) % 2]).wait()
        store(n_chunks - 1, bufs[(n_chunks - 1) % 2]).wait()

    return k(embeddings, y)


def kernel(y, embeddings):
    batch = y.shape[0]
    dim = embeddings.shape[1]
    return _embed_lookup(y.astype(jnp.int32), embeddings, batch, dim)


# speedup vs baseline: 1.0347x; 1.0347x over previous
import functools

import jax
import jax.numpy as jnp
from jax import lax
from jax.experimental import pallas as pl
from jax.experimental.pallas import tpu as pltpu
from jax.experimental.pallas import tpu_sc as plsc

NUM_CORES = 2
NUM_SUBCORES = 16
NUM_WORKERS = NUM_CORES * NUM_SUBCORES


@functools.partial(jax.jit, static_argnames=("batch", "dim"))
def _embed_lookup(y, embeddings, batch, dim):
    b_per_w = batch // NUM_WORKERS
    chunk = 16
    n_bufs = 7
    n_chunks = b_per_w // chunk

    mesh = plsc.VectorSubcoreMesh(core_axis_name="c", subcore_axis_name="s")

    @functools.partial(
        pl.kernel,
        mesh=mesh,
        out_type=jax.ShapeDtypeStruct((batch, dim), jnp.float32),
        scratch_types=[
            pltpu.VMEM((b_per_w,), jnp.int32),
        ]
        + [pltpu.VMEM((chunk, dim), jnp.float32) for _ in range(n_bufs)]
        + [
            pltpu.SemaphoreType.DMA,
            pltpu.SemaphoreType.DMA,
        ],
    )
    def k(table_hbm, idx_hbm, out_hbm, idx_v, *rest):
        bufs = rest[:n_bufs]
        gsem, ssem = rest[n_bufs:]
        wid = lax.axis_index("s") * NUM_CORES + lax.axis_index("c")
        base = wid * b_per_w
        pltpu.sync_copy(idx_hbm.at[pl.ds(base, b_per_w)], idx_v)

        def gather(c, buf):
            return pltpu.make_async_copy(
                table_hbm.at[idx_v.at[pl.ds(c * chunk, chunk)]], buf, gsem
            )

        def store(c, buf):
            return pltpu.make_async_copy(
                buf, out_hbm.at[pl.ds(base + c * chunk, chunk)], ssem
            )

        for c in range(min(n_bufs, n_chunks)):
            gather(c, bufs[c % n_bufs]).start()
        for c in range(n_chunks):
            buf = bufs[c % n_bufs]
            gather(c, buf).wait()
            store(c, buf).start()
            nxt = c + n_bufs
            if nxt < n_chunks:
                store(c, buf).wait()
                gather(nxt, buf).start()
        for c in range(max(0, n_chunks - n_bufs), n_chunks):
            store(c, bufs[c % n_bufs]).wait()

    return k(embeddings, y)


def kernel(y, embeddings):
    batch = y.shape[0]
    dim = embeddings.shape[1]
    return _embed_lookup(y.astype(jnp.int32), embeddings, batch, dim)
